# baseline (speedup 1.0000x reference)
import functools

import jax
import jax.numpy as jnp
from jax.experimental import pallas as pl
from jax.experimental.pallas import tpu as pltpu


def _vq_block(z_ref, zn_ref, et_ref, e_ref, en_ref, zq_ref, acc_ref):
    i = pl.program_id(0)
    z = z_ref[...]
    scores = jnp.dot(z, et_ref[...], preferred_element_type=jnp.float32)
    dist = (zn_ref[...] - 2.0 * scores) + en_ref[...]
    lane = jax.lax.broadcasted_iota(jnp.int32, dist.shape, 1)
    mind = jnp.min(dist, axis=1, keepdims=True)
    big = dist.shape[1]
    idx = jnp.min(jnp.where(dist == mind, lane, big), axis=1)
    onehot = (lane == idx[:, None]).astype(jnp.float32)
    zq = jnp.dot(onehot, e_ref[...], preferred_element_type=jnp.float32,
                 precision=jax.lax.Precision.HIGHEST)
    zq_ref[...] = zq
    d = zq - z
    part = jnp.sum(d * d)

    @pl.when(i == 0)
    def _():
        acc_ref[0, 0] = part

    @pl.when(i != 0)
    def _():
        acc_ref[0, 0] += part


def kernel(z_e, embeddings):
    B, D, H, W = z_e.shape
    K = embeddings.shape[0]
    N = B * H * W
    NB = 256

    z_flat = jnp.transpose(z_e, (0, 2, 3, 1)).reshape(N, D)
    z_norm = jnp.sum(z_flat ** 2, axis=1, keepdims=True)
    e_norm = jnp.sum(embeddings ** 2, axis=1).reshape(1, K)
    e_t = embeddings.T

    zq_flat, acc = pl.pallas_call(
        _vq_block,
        grid=(N // NB,),
        in_specs=[
            pl.BlockSpec((NB, D), lambda i: (i, 0)),
            pl.BlockSpec((NB, 1), lambda i: (i, 0)),
            pl.BlockSpec((D, K), lambda i: (0, 0)),
            pl.BlockSpec((K, D), lambda i: (0, 0)),
            pl.BlockSpec((1, K), lambda i: (0, 0)),
        ],
        out_specs=[
            pl.BlockSpec((NB, D), lambda i: (i, 0)),
            pl.BlockSpec(memory_space=pltpu.SMEM, block_shape=(1, 1),
                         index_map=lambda i: (0, 0)),
        ],
        out_shape=[
            jax.ShapeDtypeStruct((N, D), jnp.float32),
            jax.ShapeDtypeStruct((1, 1), jnp.float32),
        ],
    )(z_flat, z_norm, e_t, embeddings, e_norm)

    z_q = zq_flat.reshape(B, H, W, D).transpose(0, 3, 1, 2)
    loss = acc[0, 0] * 1.25 / (N * D)
    return (z_q, loss)


# speedup vs baseline: 1.0261x; 1.0261x over previous
import functools

import jax
import jax.numpy as jnp
from jax.experimental import pallas as pl
from jax.experimental.pallas import tpu as pltpu


def _vq_block(z_ref, zn_ref, et_ref, e_ref, en_ref, zq_ref, acc_ref):
    i = pl.program_id(0)
    z = z_ref[...]
    scores = jnp.dot(z, et_ref[...], preferred_element_type=jnp.float32)
    dist = (zn_ref[...] - 2.0 * scores) + en_ref[...]
    lane = jax.lax.broadcasted_iota(jnp.int32, dist.shape, 1)
    mind = jnp.min(dist, axis=1, keepdims=True)
    big = dist.shape[1]
    idx = jnp.min(jnp.where(dist == mind, lane, big), axis=1)
    onehot = (lane == idx[:, None]).astype(jnp.float32)
    zq = jnp.dot(onehot, e_ref[...], preferred_element_type=jnp.float32,
                 precision=jax.lax.Precision.HIGHEST)
    zq_ref[...] = zq
    d = zq - z
    part = jnp.sum(d * d)

    @pl.when(i == 0)
    def _():
        acc_ref[0, 0] = part

    @pl.when(i != 0)
    def _():
        acc_ref[0, 0] += part


def kernel(z_e, embeddings):
    B, D, H, W = z_e.shape
    K = embeddings.shape[0]
    N = B * H * W
    NB = 512

    z_flat = jnp.transpose(z_e, (0, 2, 3, 1)).reshape(N, D)
    z_norm = jnp.sum(z_flat ** 2, axis=1, keepdims=True)
    e_norm = jnp.sum(embeddings ** 2, axis=1).reshape(1, K)
    e_t = embeddings.T

    zq_flat, acc = pl.pallas_call(
        _vq_block,
        grid=(N // NB,),
        in_specs=[
            pl.BlockSpec((NB, D), lambda i: (i, 0)),
            pl.BlockSpec((NB, 1), lambda i: (i, 0)),
            pl.BlockSpec((D, K), lambda i: (0, 0)),
            pl.BlockSpec((K, D), lambda i: (0, 0)),
            pl.BlockSpec((1, K), lambda i: (0, 0)),
        ],
        out_specs=[
            pl.BlockSpec((NB, D), lambda i: (i, 0)),
            pl.BlockSpec(memory_space=pltpu.SMEM, block_shape=(1, 1),
                         index_map=lambda i: (0, 0)),
        ],
        out_shape=[
            jax.ShapeDtypeStruct((N, D), jnp.float32),
            jax.ShapeDtypeStruct((1, 1), jnp.float32),
        ],
    )(z_flat, z_norm, e_t, embeddings, e_norm)

    z_q = zq_flat.reshape(B, H, W, D).transpose(0, 3, 1, 2)
    loss = acc[0, 0] * 1.25 / (N * D)
    return (z_q, loss)


# speedup vs baseline: 1.6445x; 1.6028x over previous
import functools

import jax
import jax.numpy as jnp
from jax.experimental import pallas as pl
from jax.experimental.pallas import tpu as pltpu


def _vq_block(z_ref, zn_ref, et_ref, eh_ref, el_ref, en_ref, zq_ref, acc_ref):
    i = pl.program_id(0)
    z = z_ref[...]
    scores = jnp.dot(z, et_ref[...], preferred_element_type=jnp.float32)
    dist = (zn_ref[...] - 2.0 * scores) + en_ref[...]
    lane = jax.lax.broadcasted_iota(jnp.int32, dist.shape, 1)
    mind = jnp.min(dist, axis=1, keepdims=True)
    big = dist.shape[1]
    idx = jnp.min(jnp.where(dist == mind, lane, big), axis=1)
    onehot = (lane == idx[:, None]).astype(jnp.bfloat16)
    zq = (jnp.dot(onehot, eh_ref[...], preferred_element_type=jnp.float32)
          + jnp.dot(onehot, el_ref[...], preferred_element_type=jnp.float32))
    zq_ref[...] = zq
    d = zq - z
    part = jnp.sum(d * d)

    @pl.when(i == 0)
    def _():
        acc_ref[0, 0] = part

    @pl.when(i != 0)
    def _():
        acc_ref[0, 0] += part


def kernel(z_e, embeddings):
    B, D, H, W = z_e.shape
    K = embeddings.shape[0]
    N = B * H * W
    NB = 512

    z_flat = jnp.transpose(z_e, (0, 2, 3, 1)).reshape(N, D)
    z_norm = jnp.sum(z_flat ** 2, axis=1, keepdims=True)
    e_norm = jnp.sum(embeddings ** 2, axis=1).reshape(1, K)
    e_t = embeddings.T
    e_hi = embeddings.astype(jnp.bfloat16)
    e_lo = (embeddings - e_hi.astype(jnp.float32)).astype(jnp.bfloat16)

    zq_flat, acc = pl.pallas_call(
        _vq_block,
        grid=(N // NB,),
        in_specs=[
            pl.BlockSpec((NB, D), lambda i: (i, 0)),
            pl.BlockSpec((NB, 1), lambda i: (i, 0)),
            pl.BlockSpec((D, K), lambda i: (0, 0)),
            pl.BlockSpec((K, D), lambda i: (0, 0)),
            pl.BlockSpec((K, D), lambda i: (0, 0)),
            pl.BlockSpec((1, K), lambda i: (0, 0)),
        ],
        out_specs=[
            pl.BlockSpec((NB, D), lambda i: (i, 0)),
            pl.BlockSpec(memory_space=pltpu.SMEM, block_shape=(1, 1),
                         index_map=lambda i: (0, 0)),
        ],
        out_shape=[
            jax.ShapeDtypeStruct((N, D), jnp.float32),
            jax.ShapeDtypeStruct((1, 1), jnp.float32),
        ],
    )(z_flat, z_norm, e_t, e_hi, e_lo, e_norm)

    z_q = zq_flat.reshape(B, H, W, D).transpose(0, 3, 1, 2)
    loss = acc[0, 0] * 1.25 / (N * D)
    return (z_q, loss)
